# baseline (speedup 1.0000x reference)
import functools

import jax
import jax.numpy as jnp
from jax import lax
from jax.experimental import pallas as pl
from jax.experimental.pallas import tpu as pltpu
from jax.experimental.pallas import tpu_sc as plsc

VOCAB = 100000
D = 64
BATCH = 1024
STATES = 20
ATOMS = 20
PER_ROW = ATOMS * 3
ROWS = BATCH * (1 + STATES)
NC = 2
NS = 16
NW = NC * NS
OBS_PER_W = BATCH // NW
ACT_PER_W = BATCH * STATES // NW
ROWS_PER_W = OBS_PER_W + ACT_PER_W
R_BLK = 2
IDX_BLK = R_BLK * PER_ROW
N_BLK = ROWS_PER_W // R_BLK
OBS_BLK = OBS_PER_W // R_BLK
NBUF = 4
LANES = 16
NCH = D // LANES
J_GRP = 15
SCALE = 1.0 / ATOMS


@functools.partial(
    pl.kernel,
    mesh=plsc.VectorSubcoreMesh(core_axis_name="c", subcore_axis_name="s"),
    out_type=jax.ShapeDtypeStruct((ROWS * D,), jnp.float32),
    compiler_params=pltpu.CompilerParams(use_tc_tiling_on_sc=False),
    scratch_types=[
        pltpu.VMEM((ROWS_PER_W * PER_ROW,), jnp.int32),
        [pltpu.VMEM((IDX_BLK, D), jnp.float32) for _ in range(NBUF)],
        [pltpu.VMEM((R_BLK * D,), jnp.float32) for _ in range(NBUF)],
        [pltpu.SemaphoreType.DMA for _ in range(NBUF)],
        [pltpu.SemaphoreType.DMA for _ in range(NBUF)],
    ],
)
def _pooled_lookup(obs_idx_hbm, act_idx_hbm, table_hbm, out_hbm, idx_all,
                   rows_bufs, out_bufs, semg, semo):
    wid = lax.axis_index("s") * NC + lax.axis_index("c")

    pltpu.sync_copy(
        obs_idx_hbm.at[pl.ds(wid * OBS_PER_W * PER_ROW, OBS_PER_W * PER_ROW)],
        idx_all.at[pl.ds(0, OBS_PER_W * PER_ROW)])
    pltpu.sync_copy(
        act_idx_hbm.at[pl.ds(wid * ACT_PER_W * PER_ROW, ACT_PER_W * PER_ROW)],
        idx_all.at[pl.ds(OBS_PER_W * PER_ROW, ACT_PER_W * PER_ROW)])

    def gather(i, rows_b, sem_b):
        return pltpu.make_async_copy(
            table_hbm.at[idx_all.at[pl.ds(i * IDX_BLK, IDX_BLK)]],
            rows_b, sem_b)

    def out_store(i, out_b, sem_b):
        off = jnp.where(
            i < OBS_BLK,
            wid * (OBS_PER_W * D) + i * (R_BLK * D),
            BATCH * D + wid * (ACT_PER_W * D) + (i - OBS_BLK) * (R_BLK * D))
        return pltpu.make_async_copy(
            out_b, out_hbm.at[pl.ds(off, R_BLK * D)], sem_b)

    for b in range(NBUF):
        gather(b, rows_bufs[b], semg[b]).start()

    zeros = jnp.zeros((LANES,), jnp.float32)

    def body(p, carry):
        for b in range(NBUF):
            rows_b, out_b, semg_b, semo_b = (
                rows_bufs[b], out_bufs[b], semg[b], semo[b])
            i = NBUF * p + b
            gather(i, rows_b, semg_b).wait()

            def jbody(jj, accs):
                accs = list(accs)
                for u in range(J_GRP):
                    for r in range(R_BLK):
                        row = r * PER_ROW + jj * J_GRP + u
                        for c in range(NCH):
                            accs[r * NCH + c] = accs[r * NCH + c] + (
                                rows_b[row, pl.ds(c * LANES, LANES)])
                return tuple(accs)

            accs = lax.fori_loop(0, PER_ROW // J_GRP, jbody,
                                 (zeros,) * (R_BLK * NCH))

            @pl.when(i + NBUF < N_BLK)
            def _():
                gather(i + NBUF, rows_b, semg_b).start()

            @pl.when(i >= NBUF)
            def _():
                out_store(i, out_b, semo_b).wait()

            for r in range(R_BLK):
                for c in range(NCH):
                    out_b[pl.ds(r * D + c * LANES, LANES)] = (
                        accs[r * NCH + c] * SCALE)
            out_store(i, out_b, semo_b).start()
        return carry

    lax.fori_loop(0, N_BLK // NBUF, body, 0)
    for b in range(NBUF):
        out_store(N_BLK - NBUF + b, out_bufs[b], semo[b]).wait()


def kernel(sub_index, derived_sub_indices, action_mask, table):
    obs_idx = sub_index.reshape(BATCH * PER_ROW)
    act_idx = derived_sub_indices.reshape(BATCH * STATES * PER_ROW)
    out = _pooled_lookup(obs_idx, act_idx, table).reshape(ROWS, D)
    obs = out[:BATCH]
    act = out[BATCH:].reshape(BATCH, STATES, D)
    return (obs, act, action_mask)


# speedup vs baseline: 1.0461x; 1.0461x over previous
import functools

import jax
import jax.numpy as jnp
from jax import lax
from jax.experimental import pallas as pl
from jax.experimental.pallas import tpu as pltpu
from jax.experimental.pallas import tpu_sc as plsc

VOCAB = 100000
D = 64
BATCH = 1024
STATES = 20
ATOMS = 20
PER_ROW = ATOMS * 3
ROWS = BATCH * (1 + STATES)
NC = 2
NS = 16
NW = NC * NS
OBS_PER_W = BATCH // NW
ACT_PER_W = BATCH * STATES // NW
ROWS_PER_W = OBS_PER_W + ACT_PER_W
R_BLK = 2
IDX_BLK = R_BLK * PER_ROW
N_BLK = ROWS_PER_W // R_BLK
OBS_BLK = OBS_PER_W // R_BLK
NBUF = 4
LANES = 16
NCH = D // LANES
J_GRP = 15
SCALE = 1.0 / ATOMS


@functools.partial(
    pl.kernel,
    mesh=plsc.VectorSubcoreMesh(core_axis_name="c", subcore_axis_name="s"),
    out_type=(jax.ShapeDtypeStruct((BATCH, D), jnp.float32),
              jax.ShapeDtypeStruct((BATCH * STATES, D), jnp.float32)),
    compiler_params=pltpu.CompilerParams(use_tc_tiling_on_sc=False),
    scratch_types=[
        pltpu.VMEM((ROWS_PER_W * PER_ROW,), jnp.int32),
        [pltpu.VMEM((IDX_BLK, D), jnp.float32) for _ in range(NBUF)],
        [pltpu.VMEM((R_BLK, D), jnp.float32) for _ in range(NBUF)],
        [pltpu.SemaphoreType.DMA for _ in range(NBUF)],
        [pltpu.SemaphoreType.DMA for _ in range(NBUF)],
    ],
)
def _pooled_lookup(obs_idx_hbm, act_idx_hbm, table_hbm, obs_out_hbm,
                   act_out_hbm, idx_all, rows_bufs, out_bufs, semg, semo):
    wid = lax.axis_index("s") * NC + lax.axis_index("c")

    pltpu.sync_copy(
        obs_idx_hbm.at[pl.ds(wid * OBS_PER_W * PER_ROW, OBS_PER_W * PER_ROW)],
        idx_all.at[pl.ds(0, OBS_PER_W * PER_ROW)])
    pltpu.sync_copy(
        act_idx_hbm.at[pl.ds(wid * ACT_PER_W * PER_ROW, ACT_PER_W * PER_ROW)],
        idx_all.at[pl.ds(OBS_PER_W * PER_ROW, ACT_PER_W * PER_ROW)])

    def gather(i, rows_b, sem_b):
        return pltpu.make_async_copy(
            table_hbm.at[idx_all.at[pl.ds(i * IDX_BLK, IDX_BLK)]],
            rows_b, sem_b)

    def start_out_store(i, out_b, sem_b):
        @pl.when(i < OBS_BLK)
        def _():
            pltpu.make_async_copy(
                out_b,
                obs_out_hbm.at[pl.ds(wid * OBS_PER_W + i * R_BLK, R_BLK)],
                sem_b).start()

        @pl.when(i >= OBS_BLK)
        def _():
            pltpu.make_async_copy(
                out_b,
                act_out_hbm.at[
                    pl.ds(wid * ACT_PER_W + (i - OBS_BLK) * R_BLK, R_BLK)],
                sem_b).start()

    def wait_out_store(out_b, sem_b):
        pltpu.make_async_copy(
            out_b, obs_out_hbm.at[pl.ds(0, R_BLK)], sem_b).wait()

    for b in range(NBUF):
        gather(b, rows_bufs[b], semg[b]).start()

    zeros = jnp.zeros((LANES,), jnp.float32)

    def body(p, carry):
        for b in range(NBUF):
            rows_b, out_b, semg_b, semo_b = (
                rows_bufs[b], out_bufs[b], semg[b], semo[b])
            i = NBUF * p + b
            gather(i, rows_b, semg_b).wait()

            def jbody(jj, accs):
                accs = list(accs)
                for u in range(J_GRP):
                    for r in range(R_BLK):
                        row = r * PER_ROW + jj * J_GRP + u
                        for c in range(NCH):
                            accs[r * NCH + c] = accs[r * NCH + c] + (
                                rows_b[row, pl.ds(c * LANES, LANES)])
                return tuple(accs)

            accs = lax.fori_loop(0, PER_ROW // J_GRP, jbody,
                                 (zeros,) * (R_BLK * NCH))

            @pl.when(i + NBUF < N_BLK)
            def _():
                gather(i + NBUF, rows_b, semg_b).start()

            @pl.when(i >= NBUF)
            def _():
                wait_out_store(out_b, semo_b)

            for r in range(R_BLK):
                for c in range(NCH):
                    out_b[r, pl.ds(c * LANES, LANES)] = (
                        accs[r * NCH + c] * SCALE)
            start_out_store(i, out_b, semo_b)
        return carry

    lax.fori_loop(0, N_BLK // NBUF, body, 0)
    for b in range(NBUF):
        wait_out_store(out_bufs[b], semo[b])


def kernel(sub_index, derived_sub_indices, action_mask, table):
    obs_idx = sub_index.reshape(BATCH * PER_ROW)
    act_idx = derived_sub_indices.reshape(BATCH * STATES * PER_ROW)
    obs, act = _pooled_lookup(obs_idx, act_idx, table)
    return (obs, act.reshape(BATCH, STATES, D), action_mask)


# speedup vs baseline: 1.7101x; 1.6348x over previous
import functools

import jax
import jax.numpy as jnp
from jax import lax
from jax.experimental import pallas as pl
from jax.experimental.pallas import tpu as pltpu
from jax.experimental.pallas import tpu_sc as plsc

VOCAB = 100000
D = 64
BATCH = 1024
STATES = 20
ATOMS = 20
PER_ROW = ATOMS * 3
ROWS = BATCH * (1 + STATES)
NC = 2
NS = 16
NW = NC * NS
OBS_PER_W = BATCH // NW
ACT_PER_W = BATCH * STATES // NW
ROWS_PER_W = OBS_PER_W + ACT_PER_W
R_BLK = 2
IDX_BLK = R_BLK * PER_ROW
N_BLK = ROWS_PER_W // R_BLK
OBS_BLK = OBS_PER_W // R_BLK
NBUF = 4
LANES = 16
NCH = D // LANES
J_GRP = 15
SCALE = 1.0 / ATOMS


@functools.partial(
    pl.kernel,
    mesh=plsc.VectorSubcoreMesh(core_axis_name="c", subcore_axis_name="s"),
    out_type=(jax.ShapeDtypeStruct((BATCH, D), jnp.float32),
              jax.ShapeDtypeStruct((BATCH * STATES, D), jnp.float32)),
    compiler_params=pltpu.CompilerParams(use_tc_tiling_on_sc=False),
    scratch_types=[
        pltpu.VMEM((ROWS_PER_W * PER_ROW,), jnp.int32),
        [pltpu.VMEM((IDX_BLK, D), jnp.float32) for _ in range(NBUF)],
        [pltpu.VMEM((R_BLK, D), jnp.float32) for _ in range(NBUF)],
        [pltpu.SemaphoreType.DMA for _ in range(NBUF)],
        [pltpu.SemaphoreType.DMA for _ in range(NBUF)],
    ],
)
def _pooled_lookup(idx_hbm, table_hbm, obs_out_hbm, act_out_hbm, idx_all,
                   rows_bufs, out_bufs, semg, semo):
    wid = lax.axis_index("s") * NC + lax.axis_index("c")
    row_base = wid * ROWS_PER_W

    pltpu.sync_copy(
        idx_hbm.at[pl.ds(row_base * PER_ROW, ROWS_PER_W * PER_ROW)], idx_all)

    def gather(i, rows_b, sem_b):
        return pltpu.make_async_copy(
            table_hbm.at[idx_all.at[pl.ds(i * IDX_BLK, IDX_BLK)]],
            rows_b, sem_b)

    def start_out_store(i, out_b, sem_b):
        row0 = row_base + i * R_BLK

        @pl.when(row0 < BATCH)
        def _():
            pltpu.make_async_copy(
                out_b, obs_out_hbm.at[pl.ds(row0, R_BLK)], sem_b).start()

        @pl.when(row0 >= BATCH)
        def _():
            pltpu.make_async_copy(
                out_b, act_out_hbm.at[pl.ds(row0 - BATCH, R_BLK)],
                sem_b).start()

    def wait_out_store(out_b, sem_b):
        pltpu.make_async_copy(
            out_b, obs_out_hbm.at[pl.ds(0, R_BLK)], sem_b).wait()

    for b in range(NBUF):
        gather(b, rows_bufs[b], semg[b]).start()

    zeros = jnp.zeros((LANES,), jnp.float32)

    def body(p, carry):
        for b in range(NBUF):
            rows_b, out_b, semg_b, semo_b = (
                rows_bufs[b], out_bufs[b], semg[b], semo[b])
            i = NBUF * p + b
            gather(i, rows_b, semg_b).wait()

            def jbody(jj, accs):
                accs = list(accs)
                for u in range(J_GRP):
                    for r in range(R_BLK):
                        row = r * PER_ROW + jj * J_GRP + u
                        for c in range(NCH):
                            accs[r * NCH + c] = accs[r * NCH + c] + (
                                rows_b[row, pl.ds(c * LANES, LANES)])
                return tuple(accs)

            accs = lax.fori_loop(0, PER_ROW // J_GRP, jbody,
                                 (zeros,) * (R_BLK * NCH))

            @pl.when(i + NBUF < N_BLK)
            def _():
                gather(i + NBUF, rows_b, semg_b).start()

            @pl.when(i >= NBUF)
            def _():
                wait_out_store(out_b, semo_b)

            for r in range(R_BLK):
                for c in range(NCH):
                    out_b[r, pl.ds(c * LANES, LANES)] = (
                        accs[r * NCH + c] * SCALE)
            start_out_store(i, out_b, semo_b)
        return carry

    lax.fori_loop(0, N_BLK // NBUF, body, 0)
    for b in range(NBUF):
        wait_out_store(out_bufs[b], semo[b])


def kernel(sub_index, derived_sub_indices, action_mask, table):
    obs_idx = sub_index.reshape(BATCH, PER_ROW)
    act_idx = derived_sub_indices.reshape(BATCH * STATES, PER_ROW)
    flat_idx = jnp.concatenate([obs_idx, act_idx], axis=0).reshape(-1)
    obs, act = _pooled_lookup(flat_idx, table)
    return (obs, act.reshape(BATCH, STATES, D), action_mask)
